# Initial kernel scaffold; baseline (speedup 1.0000x reference)
import functools

import jax
import jax.numpy as jnp
from jax import lax
from jax.experimental import pallas as pl
from jax.experimental.pallas import tpu as pltpu
from jax.experimental.pallas import tpu_sc as plsc

N_NODES = 10000
D_FEAT = 128
N_EDGES = 320000

NC = 2
NS = 16
K = 125
C = 80
ROWS_PER_TILE = N_NODES // NS


def _scale_body(x_ref, n_ref, o_ref):
    o_ref[...] = x_ref[...] * n_ref[...]


def _combine_body(p0_ref, p1_ref, n_ref, o_ref):
    o_ref[...] = n_ref[...] * (p0_ref[...] + p1_ref[...])


_mesh = plsc.VectorSubcoreMesh(core_axis_name="c", subcore_axis_name="s")


@functools.partial(
    pl.kernel,
    out_type=(
        jax.ShapeDtypeStruct((N_NODES, D_FEAT), jnp.float32),
        jax.ShapeDtypeStruct((N_NODES, D_FEAT), jnp.float32),
    ),
    mesh=_mesh,
    scratch_types=[
        pltpu.VMEM((C, K), jnp.int32),
        pltpu.VMEM((C, K), jnp.int32),
        pltpu.VMEM((K, D_FEAT), jnp.float32),
        pltpu.VMEM_SHARED((N_NODES, D_FEAT), jnp.float32),
        pltpu.SemaphoreType.DMA,
    ],
)
def _scatter_kernel(xn_hbm, src_hbm, dst_hbm, p0_hbm, p1_hbm,
                    src_v, dst_v, rows_v, acc, sem):
    cid = lax.axis_index("c")
    sid = lax.axis_index("s")

    zeros = jnp.zeros((16,), jnp.float32)

    def _zrow(i, carry):
        for c8 in range(D_FEAT // 16):
            rows_v[i, pl.ds(c8 * 16, 16)] = zeros
        return carry

    lax.fori_loop(0, K, _zrow, 0)
    for b in range(ROWS_PER_TILE // K):
        pltpu.sync_copy(rows_v, acc.at[pl.ds(sid * ROWS_PER_TILE + b * K, K)])
    plsc.subcore_barrier()

    pltpu.sync_copy(src_hbm.at[cid, sid], src_v)
    pltpu.sync_copy(dst_hbm.at[cid, sid], dst_v)

    def _chunk(j, carry):
        pltpu.async_copy(xn_hbm.at[src_v.at[j]], rows_v, sem).wait()
        pltpu.sync_copy(rows_v, acc.at[dst_v.at[j]], add=True)
        return carry

    lax.fori_loop(0, C, _chunk, 0)

    plsc.subcore_barrier()

    row0 = sid * ROWS_PER_TILE

    @pl.when(cid == 0)
    def _write0():
        pltpu.sync_copy(acc.at[pl.ds(row0, ROWS_PER_TILE)],
                        p0_hbm.at[pl.ds(row0, ROWS_PER_TILE)])

    @pl.when(cid == 1)
    def _write1():
        pltpu.sync_copy(acc.at[pl.ds(row0, ROWS_PER_TILE)],
                        p1_hbm.at[pl.ds(row0, ROWS_PER_TILE)])


def kernel(x, norm, edge_index):
    ei = edge_index.astype(jnp.int32)
    src = ei[0].reshape(NC, NS, C, K)
    dst = ei[1].reshape(NC, NS, C, K)

    xn = pl.pallas_call(
        _scale_body,
        out_shape=jax.ShapeDtypeStruct((N_NODES, D_FEAT), jnp.float32),
    )(x, norm)

    p0, p1 = _scatter_kernel(xn, src, dst)

    y = pl.pallas_call(
        _combine_body,
        out_shape=jax.ShapeDtypeStruct((N_NODES, D_FEAT), jnp.float32),
    )(p0, p1, norm)
    return y


# speedup vs baseline: 28.4104x; 28.4104x over previous
import functools

import jax
import jax.numpy as jnp
from jax import lax
from jax.experimental import pallas as pl
from jax.experimental.pallas import tpu as pltpu
from jax.experimental.pallas import tpu_sc as plsc

N_NODES = 10000
D_FEAT = 128
N_EDGES = 320000

NC = 2
NS = 16
K = 125
C = 80

ROWS_MAIN = 624
ZCHUNK = 16
ROWS_TAIL_EXTRA = 16


def _scale_body(x_ref, n_ref, o_ref):
    o_ref[...] = x_ref[...] * n_ref[...]


def _combine_body(p0_ref, p1_ref, n_ref, o_ref):
    o_ref[...] = n_ref[...] * (p0_ref[...] + p1_ref[...])


_mesh = plsc.VectorSubcoreMesh(core_axis_name="c", subcore_axis_name="s")


@functools.partial(
    pl.kernel,
    out_type=(
        jax.ShapeDtypeStruct((N_NODES, D_FEAT), jnp.float32),
        jax.ShapeDtypeStruct((N_NODES, D_FEAT), jnp.float32),
    ),
    mesh=_mesh,
    scratch_types=[
        pltpu.VMEM((C, K), jnp.int32),
        pltpu.VMEM((C, K), jnp.int32),
        pltpu.VMEM((K, D_FEAT), jnp.float32),
        pltpu.VMEM((ZCHUNK, D_FEAT), jnp.float32),
        pltpu.VMEM_SHARED((N_NODES, D_FEAT), jnp.float32),
        pltpu.SemaphoreType.DMA,
    ],
)
def _scatter_kernel(xn_hbm, src_hbm, dst_hbm, p0_hbm, p1_hbm,
                    src_v, dst_v, rows_v, zbuf, acc, sem):
    cid = lax.axis_index("c")
    sid = lax.axis_index("s")

    zeros = jnp.zeros((16,), jnp.float32)

    def _zrow(i, carry):
        for c8 in range(D_FEAT // 16):
            zbuf[i, pl.ds(c8 * 16, 16)] = zeros
        return carry

    lax.fori_loop(0, ZCHUNK, _zrow, 0)
    for b in range(ROWS_MAIN // ZCHUNK):
        pltpu.sync_copy(zbuf, acc.at[pl.ds(sid * ROWS_MAIN + b * ZCHUNK,
                                           ZCHUNK)])

    @pl.when(sid == NS - 1)
    def _ztail():
        pltpu.sync_copy(zbuf.at[pl.ds(0, ROWS_TAIL_EXTRA)],
                        acc.at[pl.ds(NS * ROWS_MAIN, ROWS_TAIL_EXTRA)])

    plsc.subcore_barrier()

    pltpu.sync_copy(src_hbm.at[cid, sid], src_v)
    pltpu.sync_copy(dst_hbm.at[cid, sid], dst_v)

    def _chunk(j, carry):
        pltpu.async_copy(xn_hbm.at[src_v.at[j]], rows_v, sem).wait()
        pltpu.sync_copy(rows_v, acc.at[dst_v.at[j]], add=True)
        return carry

    lax.fori_loop(0, C, _chunk, 0)

    plsc.subcore_barrier()

    row0 = sid * ROWS_MAIN
    tail0 = NS * ROWS_MAIN

    @pl.when(cid == 0)
    def _write0():
        pltpu.sync_copy(acc.at[pl.ds(row0, ROWS_MAIN)],
                        p0_hbm.at[pl.ds(row0, ROWS_MAIN)])

        @pl.when(sid == NS - 1)
        def _tail0():
            pltpu.sync_copy(acc.at[pl.ds(tail0, ROWS_TAIL_EXTRA)],
                            p0_hbm.at[pl.ds(tail0, ROWS_TAIL_EXTRA)])

    @pl.when(cid == 1)
    def _write1():
        pltpu.sync_copy(acc.at[pl.ds(row0, ROWS_MAIN)],
                        p1_hbm.at[pl.ds(row0, ROWS_MAIN)])

        @pl.when(sid == NS - 1)
        def _tail1():
            pltpu.sync_copy(acc.at[pl.ds(tail0, ROWS_TAIL_EXTRA)],
                            p1_hbm.at[pl.ds(tail0, ROWS_TAIL_EXTRA)])


def kernel(x, norm, edge_index):
    ei = edge_index.astype(jnp.int32)
    src = ei[0].reshape(NC, NS, C, K)
    dst = ei[1].reshape(NC, NS, C, K)

    xn = pl.pallas_call(
        _scale_body,
        out_shape=jax.ShapeDtypeStruct((N_NODES, D_FEAT), jnp.float32),
    )(x, norm)

    p0, p1 = _scatter_kernel(xn, src, dst)

    y = pl.pallas_call(
        _combine_body,
        out_shape=jax.ShapeDtypeStruct((N_NODES, D_FEAT), jnp.float32),
    )(p0, p1, norm)
    return y


# speedup vs baseline: 39.8596x; 1.4030x over previous
import functools

import jax
import jax.numpy as jnp
from jax import lax
from jax.experimental import pallas as pl
from jax.experimental.pallas import tpu as pltpu
from jax.experimental.pallas import tpu_sc as plsc

N_NODES = 10000
D_FEAT = 128
N_EDGES = 320000

NC = 2
NS = 16
K = 125
C = 80
H = 40

ROWS_MAIN = 624
ZCHUNK = 16
ROWS_TAIL_EXTRA = 16


def _scale_body(x_ref, n_ref, o_ref):
    o_ref[...] = x_ref[...] * n_ref[...]


def _combine_body(p0_ref, p1_ref, n_ref, o_ref):
    o_ref[...] = n_ref[...] * (p0_ref[...] + p1_ref[...])


_mesh = plsc.VectorSubcoreMesh(core_axis_name="c", subcore_axis_name="s")


@functools.partial(
    pl.kernel,
    out_type=(
        jax.ShapeDtypeStruct((N_NODES, D_FEAT), jnp.float32),
        jax.ShapeDtypeStruct((N_NODES, D_FEAT), jnp.float32),
    ),
    mesh=_mesh,
    scratch_types=[
        pltpu.VMEM((H, K), jnp.int32),
        pltpu.VMEM((H, K), jnp.int32),
        pltpu.VMEM((2, K, D_FEAT), jnp.float32),
        pltpu.VMEM((ZCHUNK, D_FEAT), jnp.float32),
        pltpu.VMEM_SHARED((N_NODES, D_FEAT), jnp.float32),
        pltpu.SemaphoreType.DMA,
        pltpu.SemaphoreType.DMA,
    ],
)
def _scatter_kernel(xn_hbm, src_hbm, dst_hbm, p0_hbm, p1_hbm,
                    src_v, dst_v, rows_v, zbuf, acc, sem0, sem1):
    cid = lax.axis_index("c")
    sid = lax.axis_index("s")

    zeros = jnp.zeros((16,), jnp.float32)

    def _zrow(i, carry):
        for c8 in range(D_FEAT // 16):
            zbuf[i, pl.ds(c8 * 16, 16)] = zeros
        return carry

    lax.fori_loop(0, ZCHUNK, _zrow, 0)
    for b in range(ROWS_MAIN // ZCHUNK):
        pltpu.sync_copy(zbuf, acc.at[pl.ds(sid * ROWS_MAIN + b * ZCHUNK,
                                           ZCHUNK)])

    @pl.when(sid == NS - 1)
    def _ztail():
        pltpu.sync_copy(zbuf.at[pl.ds(0, ROWS_TAIL_EXTRA)],
                        acc.at[pl.ds(NS * ROWS_MAIN, ROWS_TAIL_EXTRA)])

    plsc.subcore_barrier()

    buf0 = rows_v.at[0]
    buf1 = rows_v.at[1]

    def _gather(j, buf, sem):
        return pltpu.async_copy(xn_hbm.at[src_v.at[j]], buf, sem)

    def _scatter(j, buf):
        pltpu.sync_copy(buf, acc.at[dst_v.at[j]], add=True)

    for half in range(C // H):
        pltpu.sync_copy(src_hbm.at[cid, sid].at[pl.ds(half * H, H)], src_v)
        pltpu.sync_copy(dst_hbm.at[cid, sid].at[pl.ds(half * H, H)], dst_v)

        _gather(0, buf0, sem0)

        def _pair(i, carry):
            j0 = 2 * i
            j1 = j0 + 1
            _gather(j1, buf1, sem1)
            pltpu.make_async_copy(xn_hbm.at[src_v.at[j0]], buf0, sem0).wait()
            _scatter(j0, buf0)
            _gather(j0 + 2, buf0, sem0)
            pltpu.make_async_copy(xn_hbm.at[src_v.at[j1]], buf1, sem1).wait()
            _scatter(j1, buf1)
            return carry

        lax.fori_loop(0, H // 2 - 1, _pair, 0)

        _gather(H - 1, buf1, sem1)
        pltpu.make_async_copy(xn_hbm.at[src_v.at[H - 2]], buf0, sem0).wait()
        _scatter(H - 2, buf0)
        pltpu.make_async_copy(xn_hbm.at[src_v.at[H - 1]], buf1, sem1).wait()
        _scatter(H - 1, buf1)

    plsc.subcore_barrier()

    row0 = sid * ROWS_MAIN
    tail0 = NS * ROWS_MAIN

    @pl.when(cid == 0)
    def _write0():
        pltpu.sync_copy(acc.at[pl.ds(row0, ROWS_MAIN)],
                        p0_hbm.at[pl.ds(row0, ROWS_MAIN)])

        @pl.when(sid == NS - 1)
        def _tail0():
            pltpu.sync_copy(acc.at[pl.ds(tail0, ROWS_TAIL_EXTRA)],
                            p0_hbm.at[pl.ds(tail0, ROWS_TAIL_EXTRA)])

    @pl.when(cid == 1)
    def _write1():
        pltpu.sync_copy(acc.at[pl.ds(row0, ROWS_MAIN)],
                        p1_hbm.at[pl.ds(row0, ROWS_MAIN)])

        @pl.when(sid == NS - 1)
        def _tail1():
            pltpu.sync_copy(acc.at[pl.ds(tail0, ROWS_TAIL_EXTRA)],
                            p1_hbm.at[pl.ds(tail0, ROWS_TAIL_EXTRA)])


def kernel(x, norm, edge_index):
    ei = edge_index.astype(jnp.int32)
    src = ei[0].reshape(NC, NS, C, K)
    dst = ei[1].reshape(NC, NS, C, K)

    xn = pl.pallas_call(
        _scale_body,
        out_shape=jax.ShapeDtypeStruct((N_NODES, D_FEAT), jnp.float32),
    )(x, norm)

    p0, p1 = _scatter_kernel(xn, src, dst)

    y = pl.pallas_call(
        _combine_body,
        out_shape=jax.ShapeDtypeStruct((N_NODES, D_FEAT), jnp.float32),
    )(p0, p1, norm)
    return y
